# Initial kernel scaffold; baseline (speedup 1.0000x reference)
import jax
import jax.numpy as jnp
from jax.experimental import pallas as pl


def kernel(codes, table):
    raise NotImplementedError("write your pallas kernel here")


# speedup vs baseline: 37.7154x; 37.7154x over previous
import functools

import jax
import jax.numpy as jnp
from jax import lax
from jax.experimental import pallas as pl
from jax.experimental.pallas import tpu as pltpu
from jax.experimental.pallas import tpu_sc as plsc

UPSAMPLE = 4
V = 100000
H = 64
B, Q, T = 1024, 8, 200
TOUT = T * UPSAMPLE

RS_CHUNKS = 25
RS_SUB = 8
RS_ROWS = V // (RS_CHUNKS * RS_SUB)

NW = 32
ROWS_PER_W = B // NW
ROWS_PER_CHUNK = 4
NCHUNK = ROWS_PER_W // ROWS_PER_CHUNK
CW = Q * T
NTG = (T + 15) // 16


def _rowsum_body(x_ref, o_ref):
    o_ref[...] = jnp.sum(x_ref[...], axis=3)


def _rowsum_tc(table):
    out = pl.pallas_call(
        _rowsum_body,
        grid=(RS_CHUNKS,),
        in_specs=[pl.BlockSpec((1, RS_SUB, RS_ROWS, H), lambda i: (i, 0, 0, 0))],
        out_specs=pl.BlockSpec((1, RS_SUB, RS_ROWS), lambda i: (i, 0, 0)),
        out_shape=jax.ShapeDtypeStruct((RS_CHUNKS, RS_SUB, RS_ROWS), jnp.float32),
    )(table.reshape(RS_CHUNKS, RS_SUB, RS_ROWS, H))
    return out.reshape(V)


def _sc_body(codes_hbm, rowsum_hbm, out_hbm, rowsum_v, codes_v, out_v):
    cid = lax.axis_index("c")
    sid = lax.axis_index("s")
    wid = sid * 2 + cid

    pltpu.sync_copy(rowsum_hbm, rowsum_v)

    codes_v[pl.ds(ROWS_PER_CHUNK * CW, 16)] = jnp.zeros((16,), jnp.int32)

    lane = lax.iota(jnp.int32, 16)
    lane4 = lane * UPSAMPLE
    lane_off = lane.astype(jnp.float32) * (0.001 * UPSAMPLE)
    tail_mask = lane < (T - (NTG - 1) * 16)

    def chunk_body(ch, carry):
        row0 = wid * ROWS_PER_W + ch * ROWS_PER_CHUNK
        pltpu.sync_copy(
            codes_hbm.at[pl.ds(row0 * CW, ROWS_PER_CHUNK * CW)],
            codes_v.at[pl.ds(0, ROWS_PER_CHUNK * CW)],
        )

        def row_body(r, carry2):
            rbase = r * CW
            obase = r * TOUT
            for tg in range(NTG):
                acc = jnp.zeros((16,), jnp.float32)
                for q in range(Q):
                    idx = codes_v[pl.ds(rbase + q * T + tg * 16, 16)]
                    acc = acc + plsc.load_gather(rowsum_v, [idx])
                tb = tg * 16
                for k in range(UPSAMPLE):
                    val = acc + (lane_off + (0.001 * (UPSAMPLE * tb + k)))
                    sidx = lane4 + (obase + UPSAMPLE * tb + k)
                    if tg == NTG - 1:
                        plsc.store_scatter(out_v, [sidx], val, mask=tail_mask)
                    else:
                        plsc.store_scatter(out_v, [sidx], val)
            return carry2

        lax.fori_loop(0, ROWS_PER_CHUNK, row_body, 0)
        pltpu.sync_copy(
            out_v,
            out_hbm.at[pl.ds(row0 * TOUT, ROWS_PER_CHUNK * TOUT)],
        )
        return carry

    lax.fori_loop(0, NCHUNK, chunk_body, 0)


@functools.partial(
    pl.kernel,
    out_type=jax.ShapeDtypeStruct((B * TOUT,), jnp.float32),
    mesh=plsc.VectorSubcoreMesh(core_axis_name="c", subcore_axis_name="s"),
    scratch_types=[
        pltpu.VMEM((V,), jnp.float32),
        pltpu.VMEM((ROWS_PER_CHUNK * CW + 16,), jnp.int32),
        pltpu.VMEM((ROWS_PER_CHUNK * TOUT,), jnp.float32),
    ],
    compiler_params=pltpu.CompilerParams(needs_layout_passes=False),
)
def _sc_gather(codes_hbm, rowsum_hbm, out_hbm, rowsum_v, codes_v, out_v):
    _sc_body(codes_hbm, rowsum_hbm, out_hbm, rowsum_v, codes_v, out_v)


def kernel(codes, table):
    rowsum = _rowsum_tc(table)
    out = _sc_gather(codes.reshape(B * Q * T), rowsum)
    return out.reshape(B, 1, TOUT)


# speedup vs baseline: 46.4039x; 1.2304x over previous
import functools

import jax
import jax.numpy as jnp
from jax import lax
from jax.experimental import pallas as pl
from jax.experimental.pallas import tpu as pltpu
from jax.experimental.pallas import tpu_sc as plsc

UPSAMPLE = 4
V = 100000
H = 64
B, Q, T = 1024, 8, 200
TOUT = T * UPSAMPLE

RS_CHUNKS = 25
RS_SUB = 8
RS_ROWS = V // (RS_CHUNKS * RS_SUB)

NW = 32
ROWS_PER_W = B // NW
ROWS_PER_CHUNK = 4
NCHUNK = ROWS_PER_W // ROWS_PER_CHUNK
TG_BASES = tuple(range(0, T - 16, 16)) + (T - 16,)


def _rowsum_body(x_ref, o_ref):
    o_ref[...] = jnp.sum(x_ref[...], axis=1).reshape(RS_SUB, RS_ROWS)


def _rowsum_tc(table):
    out = pl.pallas_call(
        _rowsum_body,
        grid=(RS_CHUNKS,),
        in_specs=[pl.BlockSpec((RS_SUB * RS_ROWS, H), lambda i: (i, 0))],
        out_specs=pl.BlockSpec((RS_SUB, RS_ROWS), lambda i: (i, 0)),
        out_shape=jax.ShapeDtypeStruct((RS_CHUNKS * RS_SUB, RS_ROWS), jnp.float32),
    )(table)
    return out.reshape(V)


def _sc_body(codes_hbm, rowsum_hbm, out_hbm, rowsum_v, codes_v, out_v):
    cid = lax.axis_index("c")
    sid = lax.axis_index("s")
    wid = sid * 2 + cid

    pltpu.sync_copy(rowsum_hbm, rowsum_v)

    lane = lax.iota(jnp.int32, 16)
    lane4 = lane * UPSAMPLE
    lane_off = lane.astype(jnp.float32) * (0.001 * UPSAMPLE)

    def chunk_body(ch, carry):
        row0 = wid * ROWS_PER_W + ch * ROWS_PER_CHUNK
        pltpu.sync_copy(codes_hbm.at[pl.ds(row0, ROWS_PER_CHUNK)], codes_v)

        def row_body(r, carry2):
            obase = r * TOUT
            for tb in TG_BASES:
                acc = jnp.zeros((16,), jnp.float32)
                for q in range(Q):
                    idx = codes_v[r, q, pl.ds(tb, 16)]
                    acc = acc + plsc.load_gather(rowsum_v, [idx])
                for k in range(UPSAMPLE):
                    val = acc + (lane_off + (0.001 * (UPSAMPLE * tb + k)))
                    sidx = lane4 + (obase + UPSAMPLE * tb + k)
                    plsc.store_scatter(out_v, [sidx], val)
            return carry2

        lax.fori_loop(0, ROWS_PER_CHUNK, row_body, 0)
        pltpu.sync_copy(
            out_v,
            out_hbm.at[pl.ds(row0 * TOUT, ROWS_PER_CHUNK * TOUT)],
        )
        return carry

    lax.fori_loop(0, NCHUNK, chunk_body, 0)


@functools.partial(
    pl.kernel,
    out_type=jax.ShapeDtypeStruct((B * TOUT,), jnp.float32),
    mesh=plsc.VectorSubcoreMesh(core_axis_name="c", subcore_axis_name="s"),
    scratch_types=[
        pltpu.VMEM((V,), jnp.float32),
        pltpu.VMEM((ROWS_PER_CHUNK, Q, T), jnp.int32),
        pltpu.VMEM((ROWS_PER_CHUNK * TOUT,), jnp.float32),
    ],
    compiler_params=pltpu.CompilerParams(needs_layout_passes=False),
)
def _sc_gather(codes_hbm, rowsum_hbm, out_hbm, rowsum_v, codes_v, out_v):
    _sc_body(codes_hbm, rowsum_hbm, out_hbm, rowsum_v, codes_v, out_v)


def kernel(codes, table):
    rowsum = _rowsum_tc(table)
    out = _sc_gather(codes, rowsum)
    return out.reshape(B, 1, TOUT)


# speedup vs baseline: 114.9140x; 2.4764x over previous
import functools

import jax
import jax.numpy as jnp
from jax import lax
from jax.experimental import pallas as pl
from jax.experimental.pallas import tpu as pltpu
from jax.experimental.pallas import tpu_sc as plsc

UPSAMPLE = 4
V = 100000
H = 64
B, Q, T = 1024, 8, 200
TOUT = T * UPSAMPLE

RS_BLK = 10240
RS_GRID = -(-V // RS_BLK)

NW = 32
TT = 8
BB = 128
NTASK = (T // TT) * (B // BB)
NCB = B // BB


def _rowsum_body(x_ref, o_ref):
    o_ref[...] = jnp.sum(x_ref[...], axis=0)


def _rowsum_tc(table):
    return pl.pallas_call(
        _rowsum_body,
        grid=(RS_GRID,),
        in_specs=[pl.BlockSpec((H, RS_BLK), lambda i: (0, i))],
        out_specs=pl.BlockSpec((RS_BLK,), lambda i: (i,)),
        out_shape=jax.ShapeDtypeStruct((V,), jnp.float32),
    )(table.T)


def _sc_body(codes_hbm, rowsum_hbm, out_hbm, rowsum_v, codes_v, out_v):
    cid = lax.axis_index("c")
    sid = lax.axis_index("s")
    wid = sid * 2 + cid

    pltpu.sync_copy(rowsum_hbm, rowsum_v)

    tk0 = (wid * NTASK) // NW
    tk1 = ((wid + 1) * NTASK) // NW

    def task_body(tk, carry):
        tb = tk // NCB
        cb = tk % NCB
        pltpu.sync_copy(
            codes_hbm.at[:, pl.ds(tb * TT, TT), pl.ds(cb * BB, BB)],
            codes_v,
        )

        def bg_body(bg, carry2):
            b16 = bg * 16
            for tt in range(TT):
                acc = jnp.zeros((16,), jnp.float32)
                for q in range(Q):
                    idx = codes_v[q, tt, pl.ds(b16, 16)]
                    acc = acc + plsc.load_gather(rowsum_v, [idx])
                off = ((tb * TT + tt) * UPSAMPLE).astype(jnp.float32) * 0.001
                for k in range(UPSAMPLE):
                    out_v[UPSAMPLE * tt + k, pl.ds(b16, 16)] = (
                        acc + (off + 0.001 * k)
                    )
            return carry2

        lax.fori_loop(0, BB // 16, bg_body, 0)
        pltpu.sync_copy(
            out_v,
            out_hbm.at[pl.ds(tb * TT * UPSAMPLE, TT * UPSAMPLE),
                       pl.ds(cb * BB, BB)],
        )
        return carry

    lax.fori_loop(tk0, tk1, task_body, 0)


@functools.partial(
    pl.kernel,
    out_type=jax.ShapeDtypeStruct((TOUT, B), jnp.float32),
    mesh=plsc.VectorSubcoreMesh(core_axis_name="c", subcore_axis_name="s"),
    scratch_types=[
        pltpu.VMEM((V,), jnp.float32),
        pltpu.VMEM((Q, TT, BB), jnp.int32),
        pltpu.VMEM((UPSAMPLE * TT, BB), jnp.float32),
    ],
    compiler_params=pltpu.CompilerParams(needs_layout_passes=False),
)
def _sc_gather(codes_hbm, rowsum_hbm, out_hbm, rowsum_v, codes_v, out_v):
    _sc_body(codes_hbm, rowsum_hbm, out_hbm, rowsum_v, codes_v, out_v)


def kernel(codes, table):
    rowsum = _rowsum_tc(table)
    out_t = _sc_gather(codes.transpose(1, 2, 0), rowsum)
    return out_t.T.reshape(B, 1, TOUT)


# speedup vs baseline: 123.4292x; 1.0741x over previous
import functools

import jax
import jax.numpy as jnp
from jax import lax
from jax.experimental import pallas as pl
from jax.experimental.pallas import tpu as pltpu
from jax.experimental.pallas import tpu_sc as plsc

UPSAMPLE = 4
V = 100000
H = 64
B, Q, T = 1024, 8, 200
TOUT = T * UPSAMPLE

RS_BLK = 10240
RS_GRID = -(-V // RS_BLK)

NW = 32
TT = 8
BB = 128
NTASK = (T // TT) * (B // BB)
NCB = B // BB
TPW = 7


def _rowsum_body(x_ref, o_ref):
    o_ref[...] = jnp.sum(x_ref[...], axis=0)


def _rowsum_tc(table):
    return pl.pallas_call(
        _rowsum_body,
        grid=(RS_GRID,),
        in_specs=[pl.BlockSpec((H, RS_BLK), lambda i: (0, i))],
        out_specs=pl.BlockSpec((RS_BLK,), lambda i: (i,)),
        out_shape=jax.ShapeDtypeStruct((V,), jnp.float32),
    )(table.T)


def _sc_body(codes_hbm, rowsum_hbm, out_hbm, rowsum_v, codes_v, out_v,
             rs_sem, c_sems, o_sems):
    cid = lax.axis_index("c")
    sid = lax.axis_index("s")
    wid = sid * 2 + cid

    tk0 = (wid * (NTASK - TPW)) // (NW - 1)
    tbs = []
    cbs = []
    for j in range(TPW):
        tk = tk0 + j
        tbs.append(tk // NCB)
        cbs.append(tk % NCB)

    def codes_dma(j):
        return pltpu.async_copy(
            codes_hbm.at[:, pl.ds(tbs[j] * TT, TT), pl.ds(cbs[j] * BB, BB)],
            codes_v.at[j % 2],
            c_sems[j % 2],
        )

    rs_h = pltpu.async_copy(rowsum_hbm, rowsum_v, rs_sem)
    c_hs = {0: codes_dma(0)}
    rs_h.wait()

    o_hs = {}
    for j in range(TPW):
        if j + 1 < TPW:
            c_hs[j + 1] = codes_dma(j + 1)
        c_hs[j].wait()
        if j >= 2:
            o_hs[j - 2].wait()

        def bg_body(bg, carry2):
            b16 = bg * 16
            for tt in range(TT):
                acc = jnp.zeros((16,), jnp.float32)
                for q in range(Q):
                    idx = codes_v[j % 2, q, tt, pl.ds(b16, 16)]
                    acc = acc + plsc.load_gather(rowsum_v, [idx])
                off = ((tbs[j] * TT + tt) * UPSAMPLE) * 0.001
                for k in range(UPSAMPLE):
                    out_v[j % 2, UPSAMPLE * tt + k, pl.ds(b16, 16)] = (
                        acc + (off + 0.001 * k)
                    )
            return carry2

        lax.fori_loop(0, BB // 16, bg_body, 0)
        o_hs[j] = pltpu.async_copy(
            out_v.at[j % 2],
            out_hbm.at[pl.ds(tbs[j] * TT * UPSAMPLE, TT * UPSAMPLE),
                       pl.ds(cbs[j] * BB, BB)],
            o_sems[j % 2],
        )
    o_hs[TPW - 2].wait()
    o_hs[TPW - 1].wait()


@functools.partial(
    pl.kernel,
    out_type=jax.ShapeDtypeStruct((TOUT, B), jnp.float32),
    mesh=plsc.VectorSubcoreMesh(core_axis_name="c", subcore_axis_name="s"),
    scratch_types=[
        pltpu.VMEM((V,), jnp.float32),
        pltpu.VMEM((2, Q, TT, BB), jnp.int32),
        pltpu.VMEM((2, UPSAMPLE * TT, BB), jnp.float32),
        pltpu.SemaphoreType.DMA,
        pltpu.SemaphoreType.DMA,
        pltpu.SemaphoreType.DMA,
        pltpu.SemaphoreType.DMA,
        pltpu.SemaphoreType.DMA,
    ],
    compiler_params=pltpu.CompilerParams(needs_layout_passes=False),
)
def _sc_gather(codes_hbm, rowsum_hbm, out_hbm, rowsum_v, codes_v, out_v,
               rs_sem, c_sem0, c_sem1, o_sem0, o_sem1):
    _sc_body(codes_hbm, rowsum_hbm, out_hbm, rowsum_v, codes_v, out_v,
             rs_sem, [c_sem0, c_sem1], [o_sem0, o_sem1])


def kernel(codes, table):
    rowsum = _rowsum_tc(table)
    out_t = _sc_gather(codes.transpose(1, 2, 0), rowsum)
    return out_t.T.reshape(B, 1, TOUT)


# speedup vs baseline: 129.0600x; 1.0456x over previous
import functools

import jax
import jax.numpy as jnp
from jax import lax
from jax.experimental import pallas as pl
from jax.experimental.pallas import tpu as pltpu
from jax.experimental.pallas import tpu_sc as plsc

UPSAMPLE = 4
V = 100000
H = 64
B, Q, T = 1024, 8, 200
TOUT = T * UPSAMPLE

RS_BLK = 10240
RS_GRID = -(-V // RS_BLK)

NW = 32
TT = 8
BB = 128
NTASK = (T // TT) * (B // BB)
NCB = B // BB
TPW = 7


def _rowsum_body(x_ref, o_ref):
    o_ref[...] = jnp.sum(x_ref[...], axis=0)


def _rowsum_tc(table):
    return pl.pallas_call(
        _rowsum_body,
        grid=(RS_GRID,),
        in_specs=[pl.BlockSpec((H, RS_BLK), lambda i: (0, i))],
        out_specs=pl.BlockSpec((RS_BLK,), lambda i: (i,)),
        out_shape=jax.ShapeDtypeStruct((V,), jnp.float32),
    )(table.T)


def _sc_body(codes_hbm, rowsum_hbm, out_hbm, rowsum_v, codes_v, out_v,
             rs_sem, c_sems, o_sems):
    cid = lax.axis_index("c")
    sid = lax.axis_index("s")
    wid = sid * 2 + cid

    tk0 = (wid * (NTASK - TPW)) // (NW - 1)
    tbs = []
    cbs = []
    for j in range(TPW):
        tk = tk0 + j
        tbs.append(tk // NCB)
        cbs.append(tk % NCB)

    def codes_dma(j):
        return pltpu.async_copy(
            codes_hbm.at[:, pl.ds(tbs[j] * TT, TT), pl.ds(cbs[j] * BB, BB)],
            codes_v.at[j % 2],
            c_sems[j % 2],
        )

    rs_h = pltpu.async_copy(rowsum_hbm, rowsum_v, rs_sem)
    c_hs = {0: codes_dma(0)}
    rs_h.wait()

    o_hs = {}
    for j in range(TPW):
        if j + 1 < TPW:
            c_hs[j + 1] = codes_dma(j + 1)
        c_hs[j].wait()
        if j >= 2:
            o_hs[j - 2].wait()

        def bg_body(bg, carry2):
            b16 = bg * 16
            for tt in range(TT):
                g = [
                    plsc.load_gather(
                        rowsum_v, [codes_v[j % 2, q, tt, pl.ds(b16, 16)]]
                    )
                    for q in range(Q)
                ]
                while len(g) > 1:
                    g = [a + b for a, b in zip(g[::2], g[1::2])]
                off = ((tbs[j] * TT + tt) * UPSAMPLE) * 0.001
                val = g[0] + off
                for k in range(UPSAMPLE):
                    out_v[j % 2, UPSAMPLE * tt + k, pl.ds(b16, 16)] = val
                    if k + 1 < UPSAMPLE:
                        val = val + 0.001
            return carry2

        lax.fori_loop(0, BB // 16, bg_body, 0)
        o_hs[j] = pltpu.async_copy(
            out_v.at[j % 2],
            out_hbm.at[pl.ds(tbs[j] * TT * UPSAMPLE, TT * UPSAMPLE),
                       pl.ds(cbs[j] * BB, BB)],
            o_sems[j % 2],
        )
    o_hs[TPW - 2].wait()
    o_hs[TPW - 1].wait()


@functools.partial(
    pl.kernel,
    out_type=jax.ShapeDtypeStruct((TOUT, B), jnp.float32),
    mesh=plsc.VectorSubcoreMesh(core_axis_name="c", subcore_axis_name="s"),
    scratch_types=[
        pltpu.VMEM((V,), jnp.float32),
        pltpu.VMEM((2, Q, TT, BB), jnp.int32),
        pltpu.VMEM((2, UPSAMPLE * TT, BB), jnp.float32),
        pltpu.SemaphoreType.DMA,
        pltpu.SemaphoreType.DMA,
        pltpu.SemaphoreType.DMA,
        pltpu.SemaphoreType.DMA,
        pltpu.SemaphoreType.DMA,
    ],
    compiler_params=pltpu.CompilerParams(needs_layout_passes=False),
)
def _sc_gather(codes_hbm, rowsum_hbm, out_hbm, rowsum_v, codes_v, out_v,
               rs_sem, c_sem0, c_sem1, o_sem0, o_sem1):
    _sc_body(codes_hbm, rowsum_hbm, out_hbm, rowsum_v, codes_v, out_v,
             rs_sem, [c_sem0, c_sem1], [o_sem0, o_sem1])


def kernel(codes, table):
    rowsum = _rowsum_tc(table)
    out_t = _sc_gather(codes.transpose(1, 2, 0), rowsum)
    return out_t.T.reshape(B, 1, TOUT)


# speedup vs baseline: 146.8399x; 1.1378x over previous
import functools

import jax
import jax.numpy as jnp
from jax import lax
from jax.experimental import pallas as pl
from jax.experimental.pallas import tpu as pltpu
from jax.experimental.pallas import tpu_sc as plsc

UPSAMPLE = 4
V = 100000
H = 64
B, Q, T = 1024, 8, 200
TOUT = T * UPSAMPLE

RS_BLK = 10240
RS_GRID = -(-V // RS_BLK)

NW = 32
TT = 8
BB = 128
NTASK = (T // TT) * (B // BB)
NCB = B // BB
TPW = 7


def _rowsum_body(x_ref, o_ref):
    o_ref[...] = jnp.sum(x_ref[...], axis=0)


def _rowsum_tc(table):
    return pl.pallas_call(
        _rowsum_body,
        grid=(RS_GRID,),
        in_specs=[pl.BlockSpec((H, RS_BLK), lambda i: (0, i))],
        out_specs=pl.BlockSpec((RS_BLK,), lambda i: (i,)),
        out_shape=jax.ShapeDtypeStruct((V,), jnp.float32),
    )(table.T)


def _sc_body(codes_hbm, rowsum_hbm, out_hbm, rowsum_v, codes_v, out_v,
             rowsum_sh, rs_sem, c_sems, o_sems):
    cid = lax.axis_index("c")
    sid = lax.axis_index("s")
    wid = sid * 2 + cid

    tk0 = (wid * (NTASK - TPW)) // (NW - 1)
    tbs = []
    cbs = []
    for j in range(TPW):
        tk = tk0 + j
        tbs.append(tk // NCB)
        cbs.append(tk % NCB)

    def codes_dma(j):
        return pltpu.async_copy(
            codes_hbm.at[:, pl.ds(tbs[j] * TT, TT), pl.ds(cbs[j] * BB, BB)],
            codes_v.at[j % 2],
            c_sems[j % 2],
        )

    c_hs = {0: codes_dma(0)}

    @pl.when(sid == 0)
    def _stage_shared():
        pltpu.sync_copy(rowsum_hbm, rowsum_sh)

    plsc.subcore_barrier()
    rs_h = pltpu.async_copy(rowsum_sh, rowsum_v, rs_sem)
    rs_h.wait()

    o_hs = {}
    for j in range(TPW):
        if j + 1 < TPW:
            c_hs[j + 1] = codes_dma(j + 1)
        c_hs[j].wait()
        if j >= 2:
            o_hs[j - 2].wait()

        def bg_body(bg, carry2):
            b16 = bg * 16
            for tt in range(TT):
                g = [
                    plsc.load_gather(
                        rowsum_v, [codes_v[j % 2, q, tt, pl.ds(b16, 16)]]
                    )
                    for q in range(Q)
                ]
                while len(g) > 1:
                    g = [a + b for a, b in zip(g[::2], g[1::2])]
                off = ((tbs[j] * TT + tt) * UPSAMPLE) * 0.001
                val = g[0] + off
                for k in range(UPSAMPLE):
                    out_v[j % 2, UPSAMPLE * tt + k, pl.ds(b16, 16)] = val
                    if k + 1 < UPSAMPLE:
                        val = val + 0.001
            return carry2

        lax.fori_loop(0, BB // 16, bg_body, 0)
        o_hs[j] = pltpu.async_copy(
            out_v.at[j % 2],
            out_hbm.at[pl.ds(tbs[j] * TT * UPSAMPLE, TT * UPSAMPLE),
                       pl.ds(cbs[j] * BB, BB)],
            o_sems[j % 2],
        )
    o_hs[TPW - 2].wait()
    o_hs[TPW - 1].wait()


@functools.partial(
    pl.kernel,
    out_type=jax.ShapeDtypeStruct((TOUT, B), jnp.float32),
    mesh=plsc.VectorSubcoreMesh(core_axis_name="c", subcore_axis_name="s"),
    scratch_types=[
        pltpu.VMEM((V,), jnp.float32),
        pltpu.VMEM((2, Q, TT, BB), jnp.int32),
        pltpu.VMEM((2, UPSAMPLE * TT, BB), jnp.float32),
        pltpu.VMEM_SHARED((V,), jnp.float32),
        pltpu.SemaphoreType.DMA,
        pltpu.SemaphoreType.DMA,
        pltpu.SemaphoreType.DMA,
        pltpu.SemaphoreType.DMA,
        pltpu.SemaphoreType.DMA,
    ],
    compiler_params=pltpu.CompilerParams(needs_layout_passes=False),
)
def _sc_gather(codes_hbm, rowsum_hbm, out_hbm, rowsum_v, codes_v, out_v,
               rowsum_sh, rs_sem, c_sem0, c_sem1, o_sem0, o_sem1):
    _sc_body(codes_hbm, rowsum_hbm, out_hbm, rowsum_v, codes_v, out_v,
             rowsum_sh, rs_sem, [c_sem0, c_sem1], [o_sem0, o_sem1])


def kernel(codes, table):
    rowsum = _rowsum_tc(table)
    out_t = _sc_gather(codes.transpose(1, 2, 0), rowsum)
    return out_t.T.reshape(B, 1, TOUT)
